# baseline (speedup 1.0000x reference)
import functools

import jax
import jax.numpy as jnp
from jax import lax
from jax.experimental import pallas as pl
from jax.experimental.pallas import tpu as pltpu
from jax.experimental.pallas import tpu_sc as plsc

N_QUERY = 262144
N_INTERVALS = 512
CHANNELS = 64
N_KNOTS = 513
KNOTS_PAD = 528

QGROUPS = 8
CGROUPS = 4
Q_PER_TILE = N_QUERY // QGROUPS
CH = 2048
N_CHUNKS = Q_PER_TILE // CH


def _search(knots_v, tq):
    lo = jnp.zeros((16,), jnp.int32)
    hi = jnp.full((16,), N_KNOTS, jnp.int32)
    for _ in range(10):
        mid = (lo + hi) >> 1
        km = plsc.load_gather(knots_v, [mid])
        p = km < tq
        lo = jnp.where(p, mid + 1, lo)
        hi = jnp.where(p, hi, mid)
    return jnp.clip(lo - 1, 0, N_INTERVALS - 1)


def _body(t_hbm, knots_hbm, a_hbm, b_hbm, c_hbm, d_hbm, out_hbm,
          knots_v, tab_v, t_v0, t_v1, out_v0, out_v1,
          ts0, ts1, os0, os1):
    cid = lax.axis_index("c")
    sid = lax.axis_index("s")
    wid = sid * 2 + cid
    qg = wid // CGROUPS
    cg = wid % CGROUPS
    qbase = qg * Q_PER_TILE
    csl = pl.ds(cg * 16, 16)

    pltpu.sync_copy(knots_hbm, knots_v)
    pltpu.sync_copy(a_hbm.at[:, csl], tab_v.at[pl.ds(0, N_INTERVALS)])
    pltpu.sync_copy(b_hbm.at[:, csl], tab_v.at[pl.ds(N_INTERVALS, N_INTERVALS)])
    pltpu.sync_copy(c_hbm.at[:, csl], tab_v.at[pl.ds(2 * N_INTERVALS, N_INTERVALS)])
    pltpu.sync_copy(d_hbm.at[:, csl], tab_v.at[pl.ds(3 * N_INTERVALS, N_INTERVALS)])

    lane = jnp.arange(16, dtype=jnp.int32)

    def compute(t_v, out_v):
        @plsc.parallel_loop(0, CH // 16, unroll=8)
        def _vblock(v):
            tq = t_v[pl.ds(v * 16, 16)]
            idx = _search(knots_v, tq)
            fr = tq - plsc.load_gather(knots_v, [idx])
            rb = idx + N_INTERVALS
            rc = idx + 2 * N_INTERVALS
            rd = idx + 3 * N_INTERVALS
            qrow = v * 16 + lane
            for c in range(16):
                col = (lane + c) & 15
                ga = plsc.load_gather(tab_v, [idx, col])
                gb = plsc.load_gather(tab_v, [rb, col])
                gc = plsc.load_gather(tab_v, [rc, col])
                gd = plsc.load_gather(tab_v, [rd, col])
                r = ga + fr * (gb + fr * (gc + fr * gd))
                plsc.store_scatter(out_v, [qrow, col], r)

    def pair(g, carry):
        q0 = qbase + g * 2 * CH
        q1 = q0 + CH
        h0 = pltpu.async_copy(t_hbm.at[pl.ds(q0, CH)], t_v0, ts0)
        h1 = pltpu.async_copy(t_hbm.at[pl.ds(q1, CH)], t_v1, ts1)
        h0.wait()
        compute(t_v0, out_v0)
        o0 = pltpu.async_copy(out_v0, out_hbm.at[pl.ds(q0, CH), csl], os0)
        h1.wait()
        compute(t_v1, out_v1)
        o1 = pltpu.async_copy(out_v1, out_hbm.at[pl.ds(q1, CH), csl], os1)
        o0.wait()
        o1.wait()
        return carry

    lax.fori_loop(0, N_CHUNKS // 2, pair, 0)


@jax.jit
def _spline(t, knots_p, a, b, c, d):
    mesh = plsc.VectorSubcoreMesh(core_axis_name="c", subcore_axis_name="s")
    return pl.kernel(
        _body,
        out_type=jax.ShapeDtypeStruct((N_QUERY, CHANNELS), jnp.float32),
        mesh=mesh,
        scratch_types=[
            pltpu.VMEM((KNOTS_PAD,), jnp.float32),
            pltpu.VMEM((4 * N_INTERVALS, 16), jnp.float32),
            pltpu.VMEM((CH,), jnp.float32),
            pltpu.VMEM((CH,), jnp.float32),
            pltpu.VMEM((CH, 16), jnp.float32),
            pltpu.VMEM((CH, 16), jnp.float32),
            pltpu.SemaphoreType.DMA,
            pltpu.SemaphoreType.DMA,
            pltpu.SemaphoreType.DMA,
            pltpu.SemaphoreType.DMA,
        ],
        compiler_params=pltpu.CompilerParams(
            use_tc_tiling_on_sc=False, needs_layout_passes=False
        ),
    )(t, knots_p, a, b, c, d)


def kernel(t, knots, a, b, c, d):
    knots_p = jnp.concatenate(
        [knots, jnp.full((KNOTS_PAD - N_KNOTS,), knots[-1], knots.dtype)]
    )
    return _spline(t, knots_p, a, b, c, d)


# speedup vs baseline: 1.0647x; 1.0647x over previous
import functools

import jax
import jax.numpy as jnp
from jax import lax
from jax.experimental import pallas as pl
from jax.experimental.pallas import tpu as pltpu
from jax.experimental.pallas import tpu_sc as plsc

N_QUERY = 262144
N_INTERVALS = 512
CHANNELS = 64
N_KNOTS = 513
KNOTS_PAD = 528

QGROUPS = 8
CGROUPS = 4
Q_PER_TILE = N_QUERY // QGROUPS
CH = 2048
N_CHUNKS = Q_PER_TILE // CH


def _search(knots_v, tq):
    lo = jnp.zeros((16,), jnp.int32)
    hi = jnp.full((16,), N_KNOTS, jnp.int32)
    for _ in range(10):
        mid = (lo + hi) >> 1
        km = plsc.load_gather(knots_v, [mid])
        p = km < tq
        lo = jnp.where(p, mid + 1, lo)
        hi = jnp.where(p, hi, mid)
    return jnp.clip(lo - 1, 0, N_INTERVALS - 1)


def _body(t_hbm, knots_hbm, a_hbm, b_hbm, c_hbm, d_hbm, out_hbm,
          knots_v, tab_v, t_v0, t_v1, out_v0, out_v1,
          ts0, ts1, os0, os1):
    cid = lax.axis_index("c")
    sid = lax.axis_index("s")
    wid = sid * 2 + cid
    qg = wid // CGROUPS
    cg = wid % CGROUPS
    qbase = qg * Q_PER_TILE
    csl = pl.ds(cg * 16, 16)

    pltpu.sync_copy(knots_hbm, knots_v)
    pltpu.sync_copy(a_hbm.at[:, csl], tab_v.at[pl.ds(0, N_INTERVALS)])
    pltpu.sync_copy(b_hbm.at[:, csl], tab_v.at[pl.ds(N_INTERVALS, N_INTERVALS)])
    pltpu.sync_copy(c_hbm.at[:, csl], tab_v.at[pl.ds(2 * N_INTERVALS, N_INTERVALS)])
    pltpu.sync_copy(d_hbm.at[:, csl], tab_v.at[pl.ds(3 * N_INTERVALS, N_INTERVALS)])

    lane = jnp.arange(16, dtype=jnp.int32)

    def compute(t_v, out_v):
        @plsc.parallel_loop(0, CH // 16, unroll=4)
        def _vblock(v):
            tq = t_v[pl.ds(v * 16, 16)]
            idx = _search(knots_v, tq)
            fr = tq - plsc.load_gather(knots_v, [idx])
            fr2 = fr * fr
            rb = idx + N_INTERVALS
            rc = idx + 2 * N_INTERVALS
            rd = idx + 3 * N_INTERVALS
            qrow = v * 16 + lane
            for c in range(16):
                col = (lane + c) & 15
                ga = plsc.load_gather(tab_v, [idx, col])
                gb = plsc.load_gather(tab_v, [rb, col])
                gc = plsc.load_gather(tab_v, [rc, col])
                gd = plsc.load_gather(tab_v, [rd, col])
                r = (ga + fr * gb) + fr2 * (gc + fr * gd)
                plsc.store_scatter(out_v, [qrow, col], r)

    def pair(g, carry):
        q0 = qbase + g * 2 * CH
        q1 = q0 + CH
        h0 = pltpu.async_copy(t_hbm.at[pl.ds(q0, CH)], t_v0, ts0)
        h1 = pltpu.async_copy(t_hbm.at[pl.ds(q1, CH)], t_v1, ts1)
        h0.wait()
        compute(t_v0, out_v0)
        o0 = pltpu.async_copy(out_v0, out_hbm.at[pl.ds(q0, CH), csl], os0)
        h1.wait()
        compute(t_v1, out_v1)
        o1 = pltpu.async_copy(out_v1, out_hbm.at[pl.ds(q1, CH), csl], os1)
        o0.wait()
        o1.wait()
        return carry

    lax.fori_loop(0, N_CHUNKS // 2, pair, 0)


@jax.jit
def _spline(t, knots_p, a, b, c, d):
    mesh = plsc.VectorSubcoreMesh(core_axis_name="c", subcore_axis_name="s")
    return pl.kernel(
        _body,
        out_type=jax.ShapeDtypeStruct((N_QUERY, CHANNELS), jnp.float32),
        mesh=mesh,
        scratch_types=[
            pltpu.VMEM((KNOTS_PAD,), jnp.float32),
            pltpu.VMEM((4 * N_INTERVALS, 16), jnp.float32),
            pltpu.VMEM((CH,), jnp.float32),
            pltpu.VMEM((CH,), jnp.float32),
            pltpu.VMEM((CH, 16), jnp.float32),
            pltpu.VMEM((CH, 16), jnp.float32),
            pltpu.SemaphoreType.DMA,
            pltpu.SemaphoreType.DMA,
            pltpu.SemaphoreType.DMA,
            pltpu.SemaphoreType.DMA,
        ],
        compiler_params=pltpu.CompilerParams(
            use_tc_tiling_on_sc=False, needs_layout_passes=False
        ),
    )(t, knots_p, a, b, c, d)


def kernel(t, knots, a, b, c, d):
    knots_p = jnp.concatenate(
        [knots, jnp.full((KNOTS_PAD - N_KNOTS,), knots[-1], knots.dtype)]
    )
    return _spline(t, knots_p, a, b, c, d)


# speedup vs baseline: 1.0963x; 1.0297x over previous
import functools

import jax
import jax.numpy as jnp
from jax import lax
from jax.experimental import pallas as pl
from jax.experimental.pallas import tpu as pltpu
from jax.experimental.pallas import tpu_sc as plsc

N_QUERY = 262144
N_INTERVALS = 512
CHANNELS = 64
N_KNOTS = 513
KNOTS_PAD = 528

QGROUPS = 8
CGROUPS = 4
Q_PER_TILE = N_QUERY // QGROUPS
CH = 2048
N_CHUNKS = Q_PER_TILE // CH


def _search(knots_v, tq):
    lo = jnp.zeros((16,), jnp.int32)
    hi = jnp.full((16,), N_KNOTS, jnp.int32)
    for _ in range(10):
        mid = (lo + hi) >> 1
        km = plsc.load_gather(knots_v, [mid])
        p = km < tq
        lo = jnp.where(p, mid + 1, lo)
        hi = jnp.where(p, hi, mid)
    return jnp.clip(lo - 1, 0, N_INTERVALS - 1)


def _body(t_hbm, knots_hbm, a_hbm, b_hbm, c_hbm, d_hbm, out_hbm,
          knots_v, tab_v, t_v0, t_v1, out_v0, out_v1,
          ts0, ts1, os0, os1):
    cid = lax.axis_index("c")
    sid = lax.axis_index("s")
    wid = sid * 2 + cid
    qg = wid // CGROUPS
    cg = wid % CGROUPS
    qbase = qg * Q_PER_TILE
    csl = pl.ds(cg * 16, 16)

    pltpu.sync_copy(knots_hbm, knots_v)
    pltpu.sync_copy(a_hbm.at[:, csl], tab_v.at[pl.ds(0, N_INTERVALS)])
    pltpu.sync_copy(b_hbm.at[:, csl], tab_v.at[pl.ds(N_INTERVALS, N_INTERVALS)])
    pltpu.sync_copy(c_hbm.at[:, csl], tab_v.at[pl.ds(2 * N_INTERVALS, N_INTERVALS)])
    pltpu.sync_copy(d_hbm.at[:, csl], tab_v.at[pl.ds(3 * N_INTERVALS, N_INTERVALS)])

    lane = jnp.arange(16, dtype=jnp.int32)

    def compute(t_v, out_v):
        @plsc.parallel_loop(0, CH // 16, unroll=4)
        def _vblock(v):
            tq = t_v[pl.ds(v * 16, 16)]
            idx = _search(knots_v, tq)
            fr = tq - plsc.load_gather(knots_v, [idx])
            rb = idx + N_INTERVALS
            rc = idx + 2 * N_INTERVALS
            rd = idx + 3 * N_INTERVALS
            qrow = v * 16 + lane
            for c in range(16):
                col = (lane + c) & 15
                ga = plsc.load_gather(tab_v, [idx, col])
                gb = plsc.load_gather(tab_v, [rb, col])
                gc = plsc.load_gather(tab_v, [rc, col])
                gd = plsc.load_gather(tab_v, [rd, col])
                r = ga + fr * (gb + fr * (gc + fr * gd))
                plsc.store_scatter(out_v, [qrow, col], r)

    def pair(g, carry):
        q0 = qbase + g * 2 * CH
        q1 = q0 + CH
        h0 = pltpu.async_copy(t_hbm.at[pl.ds(q0, CH)], t_v0, ts0)
        h1 = pltpu.async_copy(t_hbm.at[pl.ds(q1, CH)], t_v1, ts1)
        h0.wait()
        compute(t_v0, out_v0)
        o0 = pltpu.async_copy(out_v0, out_hbm.at[pl.ds(q0, CH), csl], os0)
        h1.wait()
        compute(t_v1, out_v1)
        o1 = pltpu.async_copy(out_v1, out_hbm.at[pl.ds(q1, CH), csl], os1)
        o0.wait()
        o1.wait()
        return carry

    lax.fori_loop(0, N_CHUNKS // 2, pair, 0)


@jax.jit
def _spline(t, knots_p, a, b, c, d):
    mesh = plsc.VectorSubcoreMesh(core_axis_name="c", subcore_axis_name="s")
    return pl.kernel(
        _body,
        out_type=jax.ShapeDtypeStruct((N_QUERY, CHANNELS), jnp.float32),
        mesh=mesh,
        scratch_types=[
            pltpu.VMEM((KNOTS_PAD,), jnp.float32),
            pltpu.VMEM((4 * N_INTERVALS, 16), jnp.float32),
            pltpu.VMEM((CH,), jnp.float32),
            pltpu.VMEM((CH,), jnp.float32),
            pltpu.VMEM((CH, 16), jnp.float32),
            pltpu.VMEM((CH, 16), jnp.float32),
            pltpu.SemaphoreType.DMA,
            pltpu.SemaphoreType.DMA,
            pltpu.SemaphoreType.DMA,
            pltpu.SemaphoreType.DMA,
        ],
        compiler_params=pltpu.CompilerParams(
            use_tc_tiling_on_sc=False, needs_layout_passes=False
        ),
    )(t, knots_p, a, b, c, d)


def kernel(t, knots, a, b, c, d):
    knots_p = jnp.concatenate(
        [knots, jnp.full((KNOTS_PAD - N_KNOTS,), knots[-1], knots.dtype)]
    )
    return _spline(t, knots_p, a, b, c, d)


# speedup vs baseline: 1.1047x; 1.0076x over previous
import functools

import jax
import jax.numpy as jnp
from jax import lax
from jax.experimental import pallas as pl
from jax.experimental.pallas import tpu as pltpu
from jax.experimental.pallas import tpu_sc as plsc

N_QUERY = 262144
N_INTERVALS = 512
CHANNELS = 64
N_KNOTS = 513
KNOTS_PAD = 528

QGROUPS = 8
CGROUPS = 4
Q_PER_TILE = N_QUERY // QGROUPS
CH = 2048
N_CHUNKS = Q_PER_TILE // CH


def _search(knots_v, tq):
    lo = jnp.zeros((16,), jnp.int32)
    hi = jnp.full((16,), N_KNOTS, jnp.int32)
    for _ in range(10):
        mid = (lo + hi) >> 1
        km = plsc.load_gather(knots_v, [mid])
        p = km < tq
        lo = jnp.where(p, mid + 1, lo)
        hi = jnp.where(p, hi, mid)
    return jnp.clip(lo - 1, 0, N_INTERVALS - 1)


def _body(t_hbm, knots_hbm, a_hbm, b_hbm, c_hbm, d_hbm, out_hbm,
          knots_v, tab_v, t_v0, t_v1, out_v0, out_v1,
          ts0, ts1, os0, os1):
    cid = lax.axis_index("c")
    sid = lax.axis_index("s")
    wid = sid * 2 + cid
    qg = wid // CGROUPS
    cg = wid % CGROUPS
    qbase = qg * Q_PER_TILE
    csl = pl.ds(cg * 16, 16)

    pltpu.sync_copy(knots_hbm, knots_v.at[pl.ds(0, N_KNOTS)])
    pltpu.sync_copy(a_hbm.at[:, csl], tab_v.at[pl.ds(0, N_INTERVALS)])
    pltpu.sync_copy(b_hbm.at[:, csl], tab_v.at[pl.ds(N_INTERVALS, N_INTERVALS)])
    pltpu.sync_copy(c_hbm.at[:, csl], tab_v.at[pl.ds(2 * N_INTERVALS, N_INTERVALS)])
    pltpu.sync_copy(d_hbm.at[:, csl], tab_v.at[pl.ds(3 * N_INTERVALS, N_INTERVALS)])

    lane = jnp.arange(16, dtype=jnp.int32)

    def compute(t_v, out_v):
        @plsc.parallel_loop(0, CH // 16, unroll=4)
        def _vblock(v):
            tq = t_v[pl.ds(v * 16, 16)]
            idx = _search(knots_v, tq)
            fr = tq - plsc.load_gather(knots_v, [idx])
            rb = idx + N_INTERVALS
            rc = idx + 2 * N_INTERVALS
            rd = idx + 3 * N_INTERVALS
            qrow = v * 16 + lane
            for c in range(16):
                col = (lane + c) & 15
                ga = plsc.load_gather(tab_v, [idx, col])
                gb = plsc.load_gather(tab_v, [rb, col])
                gc = plsc.load_gather(tab_v, [rc, col])
                gd = plsc.load_gather(tab_v, [rd, col])
                r = ga + fr * (gb + fr * (gc + fr * gd))
                plsc.store_scatter(out_v, [qrow, col], r)

    def pair(g, carry):
        q0 = qbase + g * 2 * CH
        q1 = q0 + CH
        h0 = pltpu.async_copy(t_hbm.at[pl.ds(q0, CH)], t_v0, ts0)
        h1 = pltpu.async_copy(t_hbm.at[pl.ds(q1, CH)], t_v1, ts1)
        h0.wait()
        compute(t_v0, out_v0)
        o0 = pltpu.async_copy(out_v0, out_hbm.at[pl.ds(q0, CH), csl], os0)
        h1.wait()
        compute(t_v1, out_v1)
        o1 = pltpu.async_copy(out_v1, out_hbm.at[pl.ds(q1, CH), csl], os1)
        o0.wait()
        o1.wait()
        return carry

    lax.fori_loop(0, N_CHUNKS // 2, pair, 0)


@jax.jit
def _spline(t, knots, a, b, c, d):
    mesh = plsc.VectorSubcoreMesh(core_axis_name="c", subcore_axis_name="s")
    return pl.kernel(
        _body,
        out_type=jax.ShapeDtypeStruct((N_QUERY, CHANNELS), jnp.float32),
        mesh=mesh,
        scratch_types=[
            pltpu.VMEM((KNOTS_PAD,), jnp.float32),
            pltpu.VMEM((4 * N_INTERVALS, 16), jnp.float32),
            pltpu.VMEM((CH,), jnp.float32),
            pltpu.VMEM((CH,), jnp.float32),
            pltpu.VMEM((CH, 16), jnp.float32),
            pltpu.VMEM((CH, 16), jnp.float32),
            pltpu.SemaphoreType.DMA,
            pltpu.SemaphoreType.DMA,
            pltpu.SemaphoreType.DMA,
            pltpu.SemaphoreType.DMA,
        ],
        compiler_params=pltpu.CompilerParams(
            use_tc_tiling_on_sc=False, needs_layout_passes=False
        ),
    )(t, knots, a, b, c, d)


def kernel(t, knots, a, b, c, d):
    return _spline(t, knots, a, b, c, d)


# speedup vs baseline: 1.1055x; 1.0008x over previous
import functools

import jax
import jax.numpy as jnp
from jax import lax
from jax.experimental import pallas as pl
from jax.experimental.pallas import tpu as pltpu
from jax.experimental.pallas import tpu_sc as plsc

N_QUERY = 262144
N_INTERVALS = 512
CHANNELS = 64
N_KNOTS = 513
KNOTS_PAD = 528

QGROUPS = 8
CGROUPS = 4
Q_PER_TILE = N_QUERY // QGROUPS
CH = 2048
N_CHUNKS = Q_PER_TILE // CH


def _search(knots_v, tq):
    lo = jnp.zeros((16,), jnp.int32)
    hi = jnp.full((16,), N_KNOTS, jnp.int32)
    for _ in range(10):
        mid = (lo + hi) >> 1
        km = plsc.load_gather(knots_v, [mid])
        p = km < tq
        lo = jnp.where(p, mid + 1, lo)
        hi = jnp.where(p, hi, mid)
    return jnp.clip(lo - 1, 0, N_INTERVALS - 1)


def _body(t_hbm, knots_hbm, a_hbm, b_hbm, c_hbm, d_hbm, out_hbm,
          knots_v, tab_v, t_v0, t_v1, out_v0, out_v1,
          ts0, ts1, os0, os1):
    cid = lax.axis_index("c")
    sid = lax.axis_index("s")
    wid = sid * 2 + cid
    qg = wid // CGROUPS
    cg = wid % CGROUPS
    qbase = qg * Q_PER_TILE
    csl = pl.ds(cg * 16, 16)

    pltpu.sync_copy(knots_hbm, knots_v.at[pl.ds(0, N_KNOTS)])
    pltpu.sync_copy(a_hbm.at[:, csl], tab_v.at[pl.ds(0, N_INTERVALS)])
    pltpu.sync_copy(b_hbm.at[:, csl], tab_v.at[pl.ds(N_INTERVALS, N_INTERVALS)])
    pltpu.sync_copy(c_hbm.at[:, csl], tab_v.at[pl.ds(2 * N_INTERVALS, N_INTERVALS)])
    pltpu.sync_copy(d_hbm.at[:, csl], tab_v.at[pl.ds(3 * N_INTERVALS, N_INTERVALS)])

    lane = jnp.arange(16, dtype=jnp.int32)

    def compute(t_v, out_v):
        @plsc.parallel_loop(0, CH // 16, unroll=4)
        def _vblock(v):
            tq = t_v[pl.ds(v * 16, 16)]
            idx = _search(knots_v, tq)
            fr = tq - plsc.load_gather(knots_v, [idx])
            rb = idx + N_INTERVALS
            rc = idx + 2 * N_INTERVALS
            rd = idx + 3 * N_INTERVALS
            qrow = v * 16 + lane
            for c in range(16):
                col = (lane + c) & 15
                ga = plsc.load_gather(tab_v, [idx, col])
                gb = plsc.load_gather(tab_v, [rb, col])
                gc = plsc.load_gather(tab_v, [rc, col])
                gd = plsc.load_gather(tab_v, [rd, col])
                r = ga + fr * (gb + fr * (gc + fr * gd))
                plsc.store_scatter(out_v, [qrow, col], r)

    def pair(g, carry):
        q0 = qbase + g * 2 * CH
        q1 = q0 + CH
        h0 = pltpu.async_copy(t_hbm.at[pl.ds(q0, CH)], t_v0, ts0)
        h1 = pltpu.async_copy(t_hbm.at[pl.ds(q1, CH)], t_v1, ts1)
        h0.wait()
        compute(t_v0, out_v0)
        o0 = pltpu.async_copy(out_v0, out_hbm.at[pl.ds(q0, CH), csl], os0)
        h1.wait()
        compute(t_v1, out_v1)
        o1 = pltpu.async_copy(out_v1, out_hbm.at[pl.ds(q1, CH), csl], os1)
        o0.wait()
        o1.wait()
        return carry

    lax.fori_loop(0, N_CHUNKS // 2, pair, 0)


@jax.jit
def _spline(t, knots, a, b, c, d):
    mesh = plsc.VectorSubcoreMesh(core_axis_name="c", subcore_axis_name="s")
    return pl.kernel(
        _body,
        out_type=jax.ShapeDtypeStruct((N_QUERY, CHANNELS), jnp.float32),
        mesh=mesh,
        scratch_types=[
            pltpu.VMEM((KNOTS_PAD,), jnp.float32),
            pltpu.VMEM((4 * N_INTERVALS, 16), jnp.float32),
            pltpu.VMEM((CH,), jnp.float32),
            pltpu.VMEM((CH,), jnp.float32),
            pltpu.VMEM((CH, 16), jnp.float32),
            pltpu.VMEM((CH, 16), jnp.float32),
            pltpu.SemaphoreType.DMA,
            pltpu.SemaphoreType.DMA,
            pltpu.SemaphoreType.DMA,
            pltpu.SemaphoreType.DMA,
        ],
        compiler_params=pltpu.CompilerParams(
            use_tc_tiling_on_sc=False,
            needs_layout_passes=False,
            disable_bounds_checks=True,
        ),
    )(t, knots, a, b, c, d)


def kernel(t, knots, a, b, c, d):
    return _spline(t, knots, a, b, c, d)
